# baseline (speedup 1.0000x reference)
import jax
import jax.numpy as jnp
from jax import lax
from jax.experimental import pallas as pl
from jax.experimental.pallas import tpu as pltpu
from jax.experimental.pallas import tpu_sc as plsc

NC = 2
NS = 16
L = 16
NW = NC * NS

BATCH = 16384
D = 64
S_SC = 9728
S_TC = BATCH - S_SC
BPW = S_SC // NW
NG = BPW // L
K = 32


def _sc(vec, j):
    return jnp.squeeze(lax.slice(vec, (j,), (j + 1,)))


def _sc_body(users_hbm, items_hbm, ut_hbm, it_hbm, wb_hbm, out_hbm,
             idx_vu, idx_vi, urows, vrows, wb_v, tscr, out_v, sem):
    wid = lax.axis_index("s") * NC + lax.axis_index("c")
    base = wid * BPW

    pltpu.sync_copy(users_hbm.at[pl.ds(base, BPW)], idx_vu)
    pltpu.sync_copy(items_hbm.at[pl.ds(base, BPW)], idx_vi)
    pltpu.sync_copy(wb_hbm, wb_v)

    lane = lax.iota(jnp.int32, L)
    w0 = wb_v[pl.ds(0, L)]
    w1 = wb_v[pl.ds(L, L)]
    w2 = wb_v[pl.ds(2 * L, L)]
    w3 = wb_v[pl.ds(3 * L, L)]
    bvec = wb_v[pl.ds(4 * L, L)]

    def fire(g, _):
        uvec = idx_vu[pl.ds(g * L, L)]
        vvec = idx_vi[pl.ds(g * L, L)]
        for r in range(L):
            ru = _sc(uvec, r)
            rv = _sc(vvec, r)
            i = g * L + r
            pltpu.async_copy(ut_hbm.at[pl.ds(ru, 1), :],
                             urows.at[pl.ds(i, 1), :], sem)
            pltpu.async_copy(it_hbm.at[pl.ds(rv, 1), :],
                             vrows.at[pl.ds(i, 1), :], sem)
        return _

    lax.fori_loop(0, NG, fire, None)
    pltpu.make_async_copy(ut_hbm.at[pl.ds(0, BPW), :], urows, sem).wait()
    pltpu.make_async_copy(it_hbm.at[pl.ds(0, BPW), :], vrows, sem).wait()

    def compute(g, _):
        for r in range(L):
            row = g * L + r
            t = urows[row, pl.ds(0, L)] * vrows[row, pl.ds(0, L)] * w0
            t += urows[row, pl.ds(L, L)] * vrows[row, pl.ds(L, L)] * w1
            t += urows[row, pl.ds(2 * L, L)] * vrows[row, pl.ds(2 * L, L)] * w2
            t += urows[row, pl.ds(3 * L, L)] * vrows[row, pl.ds(3 * L, L)] * w3
            tscr[r, pl.ds(0, L)] = t
        acc = bvec
        for c in range(L):
            col = jnp.full((L,), c, jnp.int32)
            acc = acc + plsc.load_gather(tscr, [lane, col])
        out_v[pl.ds(g * L, L)] = acc
        return _

    lax.fori_loop(0, NG, compute, None)
    pltpu.sync_copy(out_v, out_hbm.at[pl.ds(base, BPW)])


def _sc_call(users, items, user_table, item_table, wb):
    mesh = plsc.VectorSubcoreMesh(
        core_axis_name="c", subcore_axis_name="s",
        num_cores=NC, num_subcores=NS)
    return pl.kernel(
        _sc_body,
        out_type=jax.ShapeDtypeStruct((S_SC,), jnp.float32),
        mesh=mesh,
        compiler_params=pltpu.CompilerParams(
            needs_layout_passes=False, use_tc_tiling_on_sc=True),
        scratch_types=[
            pltpu.VMEM((BPW,), jnp.int32),
            pltpu.VMEM((BPW,), jnp.int32),
            pltpu.VMEM((BPW, D), jnp.float32),
            pltpu.VMEM((BPW, D), jnp.float32),
            pltpu.VMEM((5 * L,), jnp.float32),
            pltpu.VMEM((L, 2 * D), jnp.float32),
            pltpu.VMEM((BPW,), jnp.float32),
            pltpu.SemaphoreType.DMA,
        ],
    )(users, items, user_table, item_table, wb)


def _tc_body(users_smem, items_smem, ut_hbm, it_hbm, w_vmem, b_smem,
             out_vmem, urows, vrows, usems, vsems):
    def step(i, _):
        slot = i % K

        @pl.when(i >= K)
        def _():
            pltpu.make_async_copy(ut_hbm.at[pl.ds(0, 1), :],
                                  urows.at[pl.ds(slot, 1), :],
                                  usems.at[slot]).wait()
            pltpu.make_async_copy(it_hbm.at[pl.ds(0, 1), :],
                                  vrows.at[pl.ds(slot, 1), :],
                                  vsems.at[slot]).wait()

        ru = users_smem[i]
        rv = items_smem[i]
        pltpu.make_async_copy(ut_hbm.at[pl.ds(ru, 1), :],
                              urows.at[pl.ds(i, 1), :], usems.at[slot]).start()
        pltpu.make_async_copy(it_hbm.at[pl.ds(rv, 1), :],
                              vrows.at[pl.ds(i, 1), :], vsems.at[slot]).start()
        return _

    lax.fori_loop(0, S_TC, step, None)
    for slot in range(K):
        pltpu.make_async_copy(ut_hbm.at[pl.ds(0, 1), :],
                              urows.at[pl.ds(slot, 1), :],
                              usems.at[slot]).wait()
        pltpu.make_async_copy(it_hbm.at[pl.ds(0, 1), :],
                              vrows.at[pl.ds(slot, 1), :],
                              vsems.at[slot]).wait()
    had = urows[...] * vrows[...] * w_vmem[...]
    out_vmem[...] = jnp.sum(had, axis=1, keepdims=True) + b_smem[0]


def _tc_call(users, items, user_table, item_table, out_w, out_b):
    return pl.pallas_call(
        _tc_body,
        out_shape=jax.ShapeDtypeStruct((S_TC, 1), jnp.float32),
        in_specs=[
            pl.BlockSpec(memory_space=pltpu.SMEM),
            pl.BlockSpec(memory_space=pltpu.SMEM),
            pl.BlockSpec(memory_space=pltpu.HBM),
            pl.BlockSpec(memory_space=pltpu.HBM),
            pl.BlockSpec(memory_space=pltpu.VMEM),
            pl.BlockSpec(memory_space=pltpu.SMEM),
        ],
        out_specs=pl.BlockSpec(memory_space=pltpu.VMEM),
        scratch_shapes=[
            pltpu.VMEM((S_TC, D), jnp.float32),
            pltpu.VMEM((S_TC, D), jnp.float32),
            pltpu.SemaphoreType.DMA((K,)),
            pltpu.SemaphoreType.DMA((K,)),
        ],
    )(users, items, user_table, item_table, out_w, out_b)


@jax.jit
def _gmf(users, items, user_table, item_table, wb, out_w, out_b):
    sc_out = _sc_call(users[:S_SC], items[:S_SC], user_table, item_table, wb)
    tc_out = _tc_call(users[S_SC:], items[S_SC:], user_table, item_table,
                      out_w, out_b)
    return jnp.concatenate([sc_out.reshape(S_SC, 1), tc_out], axis=0)


def kernel(users, items, user_table, item_table, out_w, out_b):
    users = users.astype(jnp.int32)
    items = items.astype(jnp.int32)
    wb = jnp.concatenate(
        [out_w.reshape(D), jnp.broadcast_to(out_b, (L,))]).astype(jnp.float32)
    return _gmf(users, items, user_table, item_table, wb, out_w, out_b)


# speedup vs baseline: 1.0007x; 1.0007x over previous
import jax
import jax.numpy as jnp
from jax import lax
from jax.experimental import pallas as pl
from jax.experimental.pallas import tpu as pltpu
from jax.experimental.pallas import tpu_sc as plsc

NC = 2
NS = 16
L = 16
NW = NC * NS

BATCH = 16384
D = 64
S_SC = 9728
S_TC = BATCH - S_SC
BPW = S_SC // NW
NG = BPW // L
K = 32


def _sc(vec, j):
    return jnp.squeeze(lax.slice(vec, (j,), (j + 1,)))


def _sc_body(users_hbm, items_hbm, ut_hbm, it_hbm, wb_hbm, out_hbm,
             idx_vu, idx_vi, urows, vrows, wb_v, tscr, out_v, sem):
    wid = lax.axis_index("s") * NC + lax.axis_index("c")
    base = wid * BPW

    pltpu.sync_copy(users_hbm.at[pl.ds(base, BPW)], idx_vu)
    pltpu.sync_copy(items_hbm.at[pl.ds(base, BPW)], idx_vi)
    pltpu.sync_copy(wb_hbm, wb_v)

    lane = lax.iota(jnp.int32, L)
    w0 = wb_v[pl.ds(0, L)]
    w1 = wb_v[pl.ds(L, L)]
    w2 = wb_v[pl.ds(2 * L, L)]
    w3 = wb_v[pl.ds(3 * L, L)]
    bvec = wb_v[pl.ds(4 * L, L)]

    def fire(g, _):
        uvec = idx_vu[pl.ds(g * L, L)]
        vvec = idx_vi[pl.ds(g * L, L)]
        for r in range(L):
            ru = _sc(uvec, r)
            rv = _sc(vvec, r)
            i = g * L + r
            pltpu.async_copy(ut_hbm.at[pl.ds(ru, 1), :],
                             urows.at[pl.ds(i, 1), :], sem)
            pltpu.async_copy(it_hbm.at[pl.ds(rv, 1), :],
                             vrows.at[pl.ds(i, 1), :], sem)
        return _

    lax.fori_loop(0, NG, fire, None)
    pltpu.make_async_copy(ut_hbm.at[pl.ds(0, BPW), :], urows, sem).wait()
    pltpu.make_async_copy(it_hbm.at[pl.ds(0, BPW), :], vrows, sem).wait()

    def compute(g, _):
        for r in range(L):
            row = g * L + r
            t = urows[row, pl.ds(0, L)] * vrows[row, pl.ds(0, L)] * w0
            t += urows[row, pl.ds(L, L)] * vrows[row, pl.ds(L, L)] * w1
            t += urows[row, pl.ds(2 * L, L)] * vrows[row, pl.ds(2 * L, L)] * w2
            t += urows[row, pl.ds(3 * L, L)] * vrows[row, pl.ds(3 * L, L)] * w3
            tscr[r, pl.ds(0, L)] = t
        acc = bvec
        for c in range(L):
            col = jnp.full((L,), c, jnp.int32)
            acc = acc + plsc.load_gather(tscr, [lane, col])
        out_v[pl.ds(g * L, L)] = acc
        return _

    lax.fori_loop(0, NG, compute, None)
    pltpu.sync_copy(out_v, out_hbm.at[pl.ds(base, BPW)])


def _sc_call(users, items, user_table, item_table, wb):
    mesh = plsc.VectorSubcoreMesh(
        core_axis_name="c", subcore_axis_name="s",
        num_cores=NC, num_subcores=NS)
    return pl.kernel(
        _sc_body,
        out_type=jax.ShapeDtypeStruct((S_SC,), jnp.float32),
        mesh=mesh,
        compiler_params=pltpu.CompilerParams(
            needs_layout_passes=False, use_tc_tiling_on_sc=True),
        cost_estimate=pl.CostEstimate(
            flops=10_000_000, bytes_accessed=200_000_000, transcendentals=0),
        scratch_types=[
            pltpu.VMEM((BPW,), jnp.int32),
            pltpu.VMEM((BPW,), jnp.int32),
            pltpu.VMEM((BPW, D), jnp.float32),
            pltpu.VMEM((BPW, D), jnp.float32),
            pltpu.VMEM((5 * L,), jnp.float32),
            pltpu.VMEM((L, 2 * D), jnp.float32),
            pltpu.VMEM((BPW,), jnp.float32),
            pltpu.SemaphoreType.DMA,
        ],
    )(users, items, user_table, item_table, wb)


def _tc_body(users_smem, items_smem, ut_hbm, it_hbm, w_vmem, b_smem,
             out_vmem, urows, vrows, usems, vsems):
    def step(i, _):
        slot = i % K

        @pl.when(i >= K)
        def _():
            pltpu.make_async_copy(ut_hbm.at[pl.ds(0, 1), :],
                                  urows.at[pl.ds(slot, 1), :],
                                  usems.at[slot]).wait()
            pltpu.make_async_copy(it_hbm.at[pl.ds(0, 1), :],
                                  vrows.at[pl.ds(slot, 1), :],
                                  vsems.at[slot]).wait()

        ru = users_smem[i]
        rv = items_smem[i]
        pltpu.make_async_copy(ut_hbm.at[pl.ds(ru, 1), :],
                              urows.at[pl.ds(i, 1), :], usems.at[slot]).start()
        pltpu.make_async_copy(it_hbm.at[pl.ds(rv, 1), :],
                              vrows.at[pl.ds(i, 1), :], vsems.at[slot]).start()
        return _

    lax.fori_loop(0, S_TC, step, None)
    for slot in range(K):
        pltpu.make_async_copy(ut_hbm.at[pl.ds(0, 1), :],
                              urows.at[pl.ds(slot, 1), :],
                              usems.at[slot]).wait()
        pltpu.make_async_copy(it_hbm.at[pl.ds(0, 1), :],
                              vrows.at[pl.ds(slot, 1), :],
                              vsems.at[slot]).wait()
    had = urows[...] * vrows[...] * w_vmem[...]
    out_vmem[...] = jnp.sum(had, axis=1, keepdims=True) + b_smem[0]


def _tc_call(users, items, user_table, item_table, out_w, out_b):
    return pl.pallas_call(
        _tc_body,
        out_shape=jax.ShapeDtypeStruct((S_TC, 1), jnp.float32),
        in_specs=[
            pl.BlockSpec(memory_space=pltpu.SMEM),
            pl.BlockSpec(memory_space=pltpu.SMEM),
            pl.BlockSpec(memory_space=pltpu.HBM),
            pl.BlockSpec(memory_space=pltpu.HBM),
            pl.BlockSpec(memory_space=pltpu.VMEM),
            pl.BlockSpec(memory_space=pltpu.SMEM),
        ],
        out_specs=pl.BlockSpec(memory_space=pltpu.VMEM),
        scratch_shapes=[
            pltpu.VMEM((S_TC, D), jnp.float32),
            pltpu.VMEM((S_TC, D), jnp.float32),
            pltpu.SemaphoreType.DMA((K,)),
            pltpu.SemaphoreType.DMA((K,)),
        ],
    )(users, items, user_table, item_table, out_w, out_b)


@jax.jit
def _gmf(users, items, user_table, item_table, wb, out_w, out_b):
    sc_out = _sc_call(users[:S_SC], items[:S_SC], user_table, item_table, wb)
    tc_out = _tc_call(users[S_SC:], items[S_SC:], user_table, item_table,
                      out_w, out_b)
    return jnp.concatenate([sc_out.reshape(S_SC, 1), tc_out], axis=0)


def kernel(users, items, user_table, item_table, out_w, out_b):
    users = users.astype(jnp.int32)
    items = items.astype(jnp.int32)
    wb = jnp.concatenate(
        [out_w.reshape(D), jnp.broadcast_to(out_b, (L,))]).astype(jnp.float32)
    return _gmf(users, items, user_table, item_table, wb, out_w, out_b)


# speedup vs baseline: 1.2056x; 1.2048x over previous
import jax
import jax.numpy as jnp
from jax import lax
from jax.experimental import pallas as pl
from jax.experimental.pallas import tpu as pltpu
from jax.experimental.pallas import tpu_sc as plsc

NC = 2
NS = 16
L = 16
NW = NC * NS

BATCH = 16384
D = 64
BPW = BATCH // NW
CH = 256
NCH = BPW // CH
NG = CH // L


def _sc(vec, j):
    return jnp.squeeze(lax.slice(vec, (j,), (j + 1,)))


def _gmf_body(users_hbm, items_hbm, ut_hbm, it_hbm, wb_hbm, out_hbm,
              idx_vu, idx_vi, urows, vrows, wb_v, tscr, out_v, sem):
    wid = lax.axis_index("s") * NC + lax.axis_index("c")
    base = wid * BPW

    pltpu.sync_copy(users_hbm.at[pl.ds(base, BPW)], idx_vu)
    pltpu.sync_copy(items_hbm.at[pl.ds(base, BPW)], idx_vi)
    pltpu.sync_copy(wb_hbm, wb_v)

    lane = lax.iota(jnp.int32, L)
    w0 = wb_v[pl.ds(0, L)]
    w1 = wb_v[pl.ds(L, L)]
    w2 = wb_v[pl.ds(2 * L, L)]
    w3 = wb_v[pl.ds(3 * L, L)]
    bvec = wb_v[pl.ds(4 * L, L)]

    def chunk(ci, _):
        cb = ci * CH

        def fire(g, _):
            uvec = idx_vu[pl.ds(cb + g * L, L)]
            vvec = idx_vi[pl.ds(cb + g * L, L)]
            for r in range(L):
                ru = _sc(uvec, r)
                rv = _sc(vvec, r)
                i = g * L + r
                pltpu.async_copy(ut_hbm.at[pl.ds(ru, 1), :],
                                 urows.at[pl.ds(i, 1), :], sem)
                pltpu.async_copy(it_hbm.at[pl.ds(rv, 1), :],
                                 vrows.at[pl.ds(i, 1), :], sem)
            return _

        lax.fori_loop(0, NG, fire, None)
        pltpu.make_async_copy(ut_hbm.at[pl.ds(0, CH), :], urows, sem).wait()
        pltpu.make_async_copy(it_hbm.at[pl.ds(0, CH), :], vrows, sem).wait()

        def compute(g, _):
            for r in range(L):
                row = g * L + r
                t = urows[row, pl.ds(0, L)] * vrows[row, pl.ds(0, L)] * w0
                t += urows[row, pl.ds(L, L)] * vrows[row, pl.ds(L, L)] * w1
                t += (urows[row, pl.ds(2 * L, L)] * vrows[row, pl.ds(2 * L, L)]
                      * w2)
                t += (urows[row, pl.ds(3 * L, L)] * vrows[row, pl.ds(3 * L, L)]
                      * w3)
                tscr[r, pl.ds(0, L)] = t
            acc = bvec
            for c in range(L):
                col = jnp.full((L,), c, jnp.int32)
                acc = acc + plsc.load_gather(tscr, [lane, col])
            out_v[pl.ds(cb + g * L, L)] = acc
            return _

        lax.fori_loop(0, NG, compute, None)
        return _

    lax.fori_loop(0, NCH, chunk, None)
    pltpu.sync_copy(out_v, out_hbm.at[pl.ds(base, BPW)])


@jax.jit
def _gmf(users, items, user_table, item_table, wb):
    mesh = plsc.VectorSubcoreMesh(
        core_axis_name="c", subcore_axis_name="s",
        num_cores=NC, num_subcores=NS)
    return pl.kernel(
        _gmf_body,
        out_type=jax.ShapeDtypeStruct((BATCH,), jnp.float32),
        mesh=mesh,
        compiler_params=pltpu.CompilerParams(
            needs_layout_passes=False, use_tc_tiling_on_sc=True),
        scratch_types=[
            pltpu.VMEM((BPW,), jnp.int32),
            pltpu.VMEM((BPW,), jnp.int32),
            pltpu.VMEM((CH, D), jnp.float32),
            pltpu.VMEM((CH, D), jnp.float32),
            pltpu.VMEM((5 * L,), jnp.float32),
            pltpu.VMEM((L, 2 * D), jnp.float32),
            pltpu.VMEM((BPW,), jnp.float32),
            pltpu.SemaphoreType.DMA,
        ],
    )(users, items, user_table, item_table, wb)


def kernel(users, items, user_table, item_table, out_w, out_b):
    users = users.astype(jnp.int32)
    items = items.astype(jnp.int32)
    wb = jnp.concatenate(
        [out_w.reshape(D), jnp.broadcast_to(out_b, (L,))]).astype(jnp.float32)
    out = _gmf(users, items, user_table, item_table, wb)
    return out.reshape(BATCH, 1)
